# baseline (speedup 1.0000x reference)
import jax
import jax.numpy as jnp
from jax.experimental import pallas as pl
from jax.experimental.pallas import tpu as pltpu

NBUF = 4
PRIME = 2


def _make_body(batch, seq_len, bs):
    nchunks = seq_len // bs

    def _body(w_hbm, o_hbm, buf, sem_in, sem_out):
        def start_in(c):
            cp = pltpu.make_async_copy(
                w_hbm.at[pl.ds(c * bs, bs)], buf.at[c % NBUF], sem_in.at[c % NBUF]
            )
            cp.start()
            return cp

        def start_outs(c):
            cps = []
            for b in range(batch):
                cp = pltpu.make_async_copy(
                    buf.at[c % NBUF],
                    o_hbm.at[b, pl.ds(c * bs, bs)],
                    sem_out.at[c % NBUF],
                )
                cp.start()
                cps.append(cp)
            return cps

        in_cp = {}
        outs = {}
        drained = set()
        for c in range(min(PRIME, nchunks)):
            in_cp[c] = start_in(c)
        for c in range(nchunks):
            in_cp[c].wait()
            outs[c] = start_outs(c)
            nxt = c + PRIME
            if nxt < nchunks:
                prev = nxt - NBUF
                if prev >= 0:
                    for w in outs[prev]:
                        w.wait()
                    drained.add(prev)
                in_cp[nxt] = start_in(nxt)
        for c in range(nchunks):
            if c not in drained:
                for w in outs[c]:
                    w.wait()

    return _body


def kernel(tokens, W_pos):
    batch, seq_len = tokens.shape
    d_model = W_pos.shape[1]
    bs = 256
    return pl.pallas_call(
        _make_body(batch, seq_len, bs),
        in_specs=[pl.BlockSpec(memory_space=pl.ANY)],
        out_specs=pl.BlockSpec(memory_space=pl.ANY),
        out_shape=jax.ShapeDtypeStruct((batch, seq_len, d_model), W_pos.dtype),
        scratch_shapes=[
            pltpu.VMEM((NBUF, bs, d_model), jnp.float32),
            pltpu.SemaphoreType.DMA((NBUF,)),
            pltpu.SemaphoreType.DMA((NBUF,)),
        ],
    )(W_pos)


# speedup vs baseline: 1.0294x; 1.0294x over previous
import jax
import jax.numpy as jnp
from jax.experimental import pallas as pl
from jax.experimental.pallas import tpu as pltpu

NBUF = 2
PRIME = 2


def _make_body(batch, seq_len, bs):
    nchunks = seq_len // bs

    def _body(w_hbm, o_hbm, buf, sem_in, sem_out):
        def start_in(c):
            cp = pltpu.make_async_copy(
                w_hbm.at[pl.ds(c * bs, bs)], buf.at[c % NBUF], sem_in.at[c % NBUF]
            )
            cp.start()
            return cp

        def start_outs(c):
            cps = []
            for b in range(batch):
                cp = pltpu.make_async_copy(
                    buf.at[c % NBUF],
                    o_hbm.at[b, pl.ds(c * bs, bs)],
                    sem_out.at[c % NBUF],
                )
                cp.start()
                cps.append(cp)
            return cps

        in_cp = {}
        outs = {}
        drained = set()
        for c in range(min(PRIME, nchunks)):
            in_cp[c] = start_in(c)
        for c in range(nchunks):
            in_cp[c].wait()
            outs[c] = start_outs(c)
            nxt = c + PRIME
            if nxt < nchunks:
                prev = nxt - NBUF
                if prev >= 0:
                    for w in outs[prev]:
                        w.wait()
                    drained.add(prev)
                in_cp[nxt] = start_in(nxt)
        for c in range(nchunks):
            if c not in drained:
                for w in outs[c]:
                    w.wait()

    return _body


def kernel(tokens, W_pos):
    batch, seq_len = tokens.shape
    d_model = W_pos.shape[1]
    bs = 1024
    return pl.pallas_call(
        _make_body(batch, seq_len, bs),
        in_specs=[pl.BlockSpec(memory_space=pl.ANY)],
        out_specs=pl.BlockSpec(memory_space=pl.ANY),
        out_shape=jax.ShapeDtypeStruct((batch, seq_len, d_model), W_pos.dtype),
        scratch_shapes=[
            pltpu.VMEM((NBUF, bs, d_model), jnp.float32),
            pltpu.SemaphoreType.DMA((NBUF,)),
            pltpu.SemaphoreType.DMA((NBUF,)),
        ],
    )(W_pos)


# speedup vs baseline: 1.0659x; 1.0355x over previous
import jax
import jax.numpy as jnp
from jax.experimental import pallas as pl
from jax.experimental.pallas import tpu as pltpu

NBUF = 4
PRIME = 4


def _make_body(batch, seq_len, bs):
    nchunks = seq_len // bs

    def _body(w_hbm, o_hbm, buf, sem_in, sem_out):
        def start_in(c):
            cp = pltpu.make_async_copy(
                w_hbm.at[pl.ds(c * bs, bs)], buf.at[c % NBUF], sem_in.at[c % NBUF]
            )
            cp.start()
            return cp

        def start_outs(c):
            cps = []
            for b in range(batch):
                cp = pltpu.make_async_copy(
                    buf.at[c % NBUF],
                    o_hbm.at[b, pl.ds(c * bs, bs)],
                    sem_out.at[c % NBUF],
                )
                cp.start()
                cps.append(cp)
            return cps

        in_cp = {}
        outs = {}
        drained = set()
        for c in range(min(PRIME, nchunks)):
            in_cp[c] = start_in(c)
        for c in range(nchunks):
            in_cp[c].wait()
            outs[c] = start_outs(c)
            nxt = c + PRIME
            if nxt < nchunks:
                prev = nxt - NBUF
                if prev >= 0:
                    for w in outs[prev]:
                        w.wait()
                    drained.add(prev)
                in_cp[nxt] = start_in(nxt)
        for c in range(nchunks):
            if c not in drained:
                for w in outs[c]:
                    w.wait()

    return _body


def kernel(tokens, W_pos):
    batch, seq_len = tokens.shape
    d_model = W_pos.shape[1]
    bs = 512
    return pl.pallas_call(
        _make_body(batch, seq_len, bs),
        in_specs=[pl.BlockSpec(memory_space=pl.ANY)],
        out_specs=pl.BlockSpec(memory_space=pl.ANY),
        out_shape=jax.ShapeDtypeStruct((batch, seq_len, d_model), W_pos.dtype),
        scratch_shapes=[
            pltpu.VMEM((NBUF, bs, d_model), jnp.float32),
            pltpu.SemaphoreType.DMA((NBUF,)),
            pltpu.SemaphoreType.DMA((NBUF,)),
        ],
    )(W_pos)
